# baseline (speedup 1.0000x reference)
import functools

import jax
import jax.numpy as jnp
from jax import lax
from jax.experimental import pallas as pl
from jax.experimental.pallas import tpu as pltpu
from jax.experimental.pallas import tpu_sc as plsc

_info = plsc.get_sparse_core_info()
_NC = _info.num_cores
_NS = _info.num_subcores
_NW = _NC * _NS
_L = _info.num_lanes

_CHUNK = 160
_N_STREAM = 25


@functools.partial(jax.jit, static_argnames=("v", "n", "d"))
def _sc_gather(W, idx, *, v, n, d):
    chunk = _CHUNK
    n_per_w = n // _NW
    n_chunks = n_per_w // chunk
    ns = _N_STREAM
    nf = n_chunks - ns
    mesh = plsc.VectorSubcoreMesh(core_axis_name="c", subcore_axis_name="s")

    @functools.partial(
        pl.kernel,
        mesh=mesh,
        out_type=jax.ShapeDtypeStruct((n, d), jnp.float32),
        scratch_types=[
            pltpu.VMEM((v, d), jnp.float32),
            pltpu.VMEM((n_per_w,), jnp.int32),
            pltpu.VMEM((chunk, d), jnp.float32),
            pltpu.VMEM((chunk, d), jnp.float32),
            pltpu.VMEM((chunk, d), jnp.float32),
            pltpu.VMEM((chunk, d), jnp.float32),
            pltpu.SemaphoreType.DMA,
            pltpu.SemaphoreType.DMA,
            pltpu.SemaphoreType.DMA,
            pltpu.SemaphoreType.DMA,
        ],
        compiler_params=pltpu.CompilerParams(use_tc_tiling_on_sc=False),
    )
    def k(table_hbm, idx_hbm, out_hbm, tab_v, idx_v,
          sb0, sb1, fb0, fb1, g0, g1, so, fo):
        wid = lax.axis_index("s") * _NC + lax.axis_index("c")
        base = wid * n_per_w
        pltpu.sync_copy(table_hbm, tab_v)
        pltpu.sync_copy(idx_hbm.at[pl.ds(base, n_per_w)], idx_v)

        sbufs = (sb0, sb1)
        gsems = (g0, g1)
        fbufs = (fb0, fb1)

        def sgather(i, b):
            return pltpu.make_async_copy(
                table_hbm.at[idx_v.at[pl.ds(i * chunk, chunk)]], sbufs[b], gsems[b]
            )

        def outcp(i, buf, sem):
            return pltpu.make_async_copy(
                buf, out_hbm.at[pl.ds(base + i * chunk, chunk)], sem
            )

        def fill(c, b):
            buf = fbufs[b]

            def body(q16, carry):
                vec = idx_v[pl.ds(c * chunk + q16 * _L, _L)]
                for u in range(_L):
                    q = q16 * _L + u
                    s = vec[u]
                    for kk in range(d // _L):
                        buf[q, pl.ds(kk * _L, _L)] = tab_v[s, pl.ds(kk * _L, _L)]
                return carry

            lax.fori_loop(0, chunk // _L, body, 0)

        sgather(0, 0).start()
        for t in range(max(ns, nf)):
            if t < ns:
                b = t % 2
                nb = (t + 1) % 2
                if t + 1 < ns:
                    if t > 0:
                        outcp(t - 1, sbufs[nb], so).wait()
                    sgather(t + 1, nb).start()
                sgather(t, b).wait()
                outcp(t, sbufs[b], so).start()
            if t < nf:
                fb = t % 2
                if t >= 2:
                    outcp(ns + t - 2, fbufs[fb], fo).wait()
                fill(ns + t, fb)
                outcp(ns + t, fbufs[fb], fo).start()
        outcp(ns - 2, sbufs[ns % 2], so).wait()
        outcp(ns - 1, sbufs[(ns - 1) % 2], so).wait()
        outcp(ns + nf - 2, fbufs[nf % 2], fo).wait()
        outcp(ns + nf - 1, fbufs[(nf - 1) % 2], fo).wait()

    return k(W, idx)


def kernel(input_, W):
    b, l = input_.shape
    v, d = W.shape
    n = b * l
    idx = input_.reshape(n)
    out = _sc_gather(W, idx, v=v, n=n, d=d)
    return out.reshape(b, l, d)


# speedup vs baseline: 1.0104x; 1.0104x over previous
import functools

import jax
import jax.numpy as jnp
from jax import lax
from jax.experimental import pallas as pl
from jax.experimental.pallas import tpu as pltpu
from jax.experimental.pallas import tpu_sc as plsc

_info = plsc.get_sparse_core_info()
_NC = _info.num_cores
_NS = _info.num_subcores
_NW = _NC * _NS
_L = _info.num_lanes

_CHUNK = 160
_N_STREAM = 23


@functools.partial(jax.jit, static_argnames=("v", "n", "d"))
def _sc_gather(W, idx, *, v, n, d):
    chunk = _CHUNK
    n_per_w = n // _NW
    n_chunks = n_per_w // chunk
    ns = _N_STREAM
    nf = n_chunks - ns
    mesh = plsc.VectorSubcoreMesh(core_axis_name="c", subcore_axis_name="s")

    @functools.partial(
        pl.kernel,
        mesh=mesh,
        out_type=jax.ShapeDtypeStruct((n, d), jnp.float32),
        scratch_types=[
            pltpu.VMEM((v, d), jnp.float32),
            pltpu.VMEM((n_per_w,), jnp.int32),
            pltpu.VMEM((chunk, d), jnp.float32),
            pltpu.VMEM((chunk, d), jnp.float32),
            pltpu.VMEM((chunk, d), jnp.float32),
            pltpu.VMEM((chunk, d), jnp.float32),
            pltpu.SemaphoreType.DMA,
            pltpu.SemaphoreType.DMA,
            pltpu.SemaphoreType.DMA,
            pltpu.SemaphoreType.DMA,
        ],
        compiler_params=pltpu.CompilerParams(use_tc_tiling_on_sc=False),
    )
    def k(table_hbm, idx_hbm, out_hbm, tab_v, idx_v,
          sb0, sb1, fb0, fb1, g0, g1, so, fo):
        wid = lax.axis_index("s") * _NC + lax.axis_index("c")
        base = wid * n_per_w
        pltpu.sync_copy(table_hbm, tab_v)
        pltpu.sync_copy(idx_hbm.at[pl.ds(base, n_per_w)], idx_v)

        sbufs = (sb0, sb1)
        gsems = (g0, g1)
        fbufs = (fb0, fb1)

        def sgather(i, b):
            return pltpu.make_async_copy(
                table_hbm.at[idx_v.at[pl.ds(i * chunk, chunk)]], sbufs[b], gsems[b]
            )

        def outcp(i, buf, sem):
            return pltpu.make_async_copy(
                buf, out_hbm.at[pl.ds(base + i * chunk, chunk)], sem
            )

        def fill(c, b):
            buf = fbufs[b]

            def body(q16, carry):
                vec = idx_v[pl.ds(c * chunk + q16 * _L, _L)]
                for u in range(_L):
                    q = q16 * _L + u
                    s = vec[u]
                    for kk in range(d // _L):
                        buf[q, pl.ds(kk * _L, _L)] = tab_v[s, pl.ds(kk * _L, _L)]
                return carry

            lax.fori_loop(0, chunk // _L, body, 0)

        sgather(0, 0).start()
        for t in range(max(ns, nf)):
            if t < ns:
                b = t % 2
                nb = (t + 1) % 2
                if t + 1 < ns:
                    if t > 0:
                        outcp(t - 1, sbufs[nb], so).wait()
                    sgather(t + 1, nb).start()
                sgather(t, b).wait()
                outcp(t, sbufs[b], so).start()
            if t < nf:
                fb = t % 2
                if t >= 2:
                    outcp(ns + t - 2, fbufs[fb], fo).wait()
                fill(ns + t, fb)
                outcp(ns + t, fbufs[fb], fo).start()
        outcp(ns - 2, sbufs[ns % 2], so).wait()
        outcp(ns - 1, sbufs[(ns - 1) % 2], so).wait()
        outcp(ns + nf - 2, fbufs[nf % 2], fo).wait()
        outcp(ns + nf - 1, fbufs[(nf - 1) % 2], fo).wait()

    return k(W, idx)


def kernel(input_, W):
    b, l = input_.shape
    v, d = W.shape
    n = b * l
    idx = input_.reshape(n)
    out = _sc_gather(W, idx, v=v, n=n, d=d)
    return out.reshape(b, l, d)
